# baseline (speedup 1.0000x reference)
import jax
import jax.numpy as jnp
from jax import lax
from jax.experimental import pallas as pl
from jax.experimental.pallas import tpu as pltpu
from jax.experimental.pallas import tpu_sc as plsc

B = 16384
D = 32
PK = 128 // D
NC = 2
NS = 16
NW = NC * NS
BPW = B // NW
L = 16
NCH = 8
CH = BPW // NCH


def _sc_body(uidx_hbm, pidx_hbm, u128_hbm, p128_hbm, pbias_hbm,
             umask_hbm, pmask_hbm,
             score_hbm, concat_hbm,
             uidx_v, pidx_v, qu_v, qp_v, su_v, sp_v,
             ubuf_a, ubuf_b, pbuf_a, pbuf_b,
             bias_v, umask_v, pmask_v, concat_v, score_v, tbuf_v,
             sem_u, sem_p, sem_b):
  c = lax.axis_index("c")
  s = lax.axis_index("s")
  wid = s * NC + c
  base = wid * BPW

  pltpu.sync_copy(uidx_hbm.at[pl.ds(base, BPW)], uidx_v)
  pltpu.sync_copy(pidx_hbm.at[pl.ds(base, BPW)], pidx_v)
  pltpu.sync_copy(umask_hbm, umask_v)
  pltpu.sync_copy(pmask_hbm, pmask_v)

  bias_cp = pltpu.async_copy(pbias_hbm.at[pidx_v], bias_v, sem_b)

  def precompute(v, carry):
    uvec = uidx_v[pl.ds(v * L, L)]
    pvec = pidx_v[pl.ds(v * L, L)]
    qu_v[pl.ds(v * L, L)] = lax.shift_right_logical(uvec, 2)
    qp_v[pl.ds(v * L, L)] = lax.shift_right_logical(pvec, 2)
    su_v[pl.ds(v * L, L)] = (uvec & 3) * D
    sp_v[pl.ds(v * L, L)] = (pvec & 3) * D
    return carry

  lax.fori_loop(0, BPW // L, precompute, 0)

  ubufs = [ubuf_a, ubuf_b]
  pbufs = [pbuf_a, pbuf_b]

  def fire(j):
    sl = pl.ds(j * CH, CH)
    return [
        pltpu.async_copy(u128_hbm.at[qu_v.at[sl]], ubufs[j % 2], sem_u),
        pltpu.async_copy(p128_hbm.at[qp_v.at[sl]], pbufs[j % 2], sem_p),
    ]

  um0 = umask_v[pl.ds(0, L)]
  um1 = umask_v[pl.ds(L, L)]
  pm0 = pmask_v[pl.ds(0, L)]
  pm1 = pmask_v[pl.ds(L, L)]
  lane = lax.iota(jnp.int32, L)
  lane17 = lane * (L + 1)

  pending = {0: fire(0)}
  bias_cp.wait()
  for j in range(NCH):
    if j + 1 < NCH:
      pending[j + 1] = fire(j + 1)
    for cp in pending.pop(j):
      cp.wait()
    ub = ubufs[j % 2]
    pb = pbufs[j % 2]

    def group(g, carry, ub=ub, pb=pb, j=j):
      goff = g * L
      for r in range(L):
        i = goff + r
        o = j * CH + i
        rfull = jnp.full((L,), r, jnp.int32)
        iv = rfull + goff
        ov = rfull + (j * CH + goff)
        cu = plsc.load_gather(su_v, [ov]) + lane
        cp_ = plsc.load_gather(sp_v, [ov]) + lane
        u0 = plsc.load_gather(ub, [iv, cu]) * um0
        u1 = plsc.load_gather(ub, [iv, cu + L]) * um1
        p0 = plsc.load_gather(pb, [iv, cp_]) * pm0
        p1 = plsc.load_gather(pb, [iv, cp_ + L]) * pm1
        concat_v[o, pl.ds(0, L)] = u0
        concat_v[o, pl.ds(L, L)] = u1
        concat_v[o, pl.ds(2 * L, L)] = p0
        concat_v[o, pl.ds(3 * L, L)] = p1
        t = u0 * p0 + u1 * p1
        plsc.store_scatter(tbuf_v, [lane17 + r], t)
      acc = tbuf_v[pl.ds(0, L)]
      for k in range(1, L):
        acc = acc + tbuf_v[pl.ds(k * (L + 1), L)]
      og = j * CH + goff
      score_v[pl.ds(og, L)] = acc + bias_v[pl.ds(og, L)]
      return carry

    lax.fori_loop(0, CH // L, group, 0)

  pltpu.sync_copy(score_v, score_hbm.at[pl.ds(base, BPW)])
  pltpu.sync_copy(concat_v, concat_hbm.at[pl.ds(base, BPW)])


@jax.jit
def _mvem_sc(uidx, pidx, u128, p128, product_bias,
             user_emb_mask, product_emb_mask):
  mesh = plsc.VectorSubcoreMesh(
      core_axis_name="c", subcore_axis_name="s", num_cores=NC,
      num_subcores=NS)
  run = pl.kernel(
      _sc_body,
      out_type=(jax.ShapeDtypeStruct((B,), jnp.float32),
                jax.ShapeDtypeStruct((B, 2 * D), jnp.float32)),
      mesh=mesh,
      scratch_types=[
          pltpu.VMEM((BPW,), jnp.int32),
          pltpu.VMEM((BPW,), jnp.int32),
          pltpu.VMEM((BPW,), jnp.int32),
          pltpu.VMEM((BPW,), jnp.int32),
          pltpu.VMEM((BPW,), jnp.int32),
          pltpu.VMEM((BPW,), jnp.int32),
          pltpu.VMEM((CH, 128), jnp.float32),
          pltpu.VMEM((CH, 128), jnp.float32),
          pltpu.VMEM((CH, 128), jnp.float32),
          pltpu.VMEM((CH, 128), jnp.float32),
          pltpu.VMEM((BPW,), jnp.float32),
          pltpu.VMEM((D,), jnp.float32),
          pltpu.VMEM((D,), jnp.float32),
          pltpu.VMEM((BPW, 2 * D), jnp.float32),
          pltpu.VMEM((BPW,), jnp.float32),
          pltpu.VMEM((L * (L + 1),), jnp.float32),
          pltpu.SemaphoreType.DMA,
          pltpu.SemaphoreType.DMA,
          pltpu.SemaphoreType.DMA,
      ],
      compiler_params=pltpu.CompilerParams(
          needs_layout_passes=False, use_tc_tiling_on_sc=True),
  )
  return run(uidx, pidx, u128, p128, product_bias,
             user_emb_mask, product_emb_mask)


def kernel(user_idx, product_idx, user_emb, product_emb, product_bias,
           user_emb_mask, product_emb_mask):
  u128 = user_emb.reshape(user_emb.shape[0] // PK, PK * D)
  p128 = product_emb.reshape(product_emb.shape[0] // PK, PK * D)
  score, concat = _mvem_sc(user_idx.astype(jnp.int32),
                           product_idx.astype(jnp.int32),
                           u128, p128, product_bias,
                           user_emb_mask, product_emb_mask)
  return score, concat


# speedup vs baseline: 1.4784x; 1.4784x over previous
import jax
import jax.numpy as jnp
from jax import lax
from jax.experimental import pallas as pl
from jax.experimental.pallas import tpu as pltpu
from jax.experimental.pallas import tpu_sc as plsc

B = 16384
D = 32
NC = 2
NS = 16
NW = NC * NS
BPW = B // NW
L = 16
NCH = 8
CH = BPW // NCH


def _sc_body(uidx_hbm, pidx_hbm, uemb_hbm, pemb_hbm, pbias_hbm,
             umask_hbm, pmask_hbm,
             score_hbm, concat_hbm,
             uidx_v, pidx_v, ubuf_v, pbuf_v,
             bias_v, umask_v, pmask_v, concat_v, score_v, tbuf_v,
             sem_u, sem_p, sem_b):
  c = lax.axis_index("c")
  s = lax.axis_index("s")
  wid = s * NC + c
  base = wid * BPW

  pltpu.sync_copy(uidx_hbm.at[pl.ds(base, BPW)], uidx_v)
  pltpu.sync_copy(pidx_hbm.at[pl.ds(base, BPW)], pidx_v)
  pltpu.sync_copy(umask_hbm, umask_v)
  pltpu.sync_copy(pmask_hbm, pmask_v)

  bias_cp = pltpu.async_copy(pbias_hbm.at[pidx_v], bias_v, sem_b)

  lane = lax.iota(jnp.int32, L)
  zero16 = jnp.zeros((L,), jnp.int32)

  def fire(j):
    @plsc.parallel_loop(0, CH, unroll=4)
    def row_fetch(i):
      off = j * CH + (i // L) * L
      sel = lane == (i % L)
      ur = jnp.sum(jnp.where(sel, uidx_v[pl.ds(off, L)], zero16))
      pr = jnp.sum(jnp.where(sel, pidx_v[pl.ds(off, L)], zero16))
      pltpu.async_copy(uemb_hbm.at[ur], ubuf_v.at[j % 2, i], sem_u)
      pltpu.async_copy(pemb_hbm.at[pr], pbuf_v.at[j % 2, i], sem_p)

  def drain(j):
    pltpu.make_async_copy(uemb_hbm.at[pl.ds(0, CH)], ubuf_v.at[j % 2],
                          sem_u).wait()
    pltpu.make_async_copy(pemb_hbm.at[pl.ds(0, CH)], pbuf_v.at[j % 2],
                          sem_p).wait()

  um0 = umask_v[pl.ds(0, L)]
  um1 = umask_v[pl.ds(L, L)]
  pm0 = pmask_v[pl.ds(0, L)]
  pm1 = pmask_v[pl.ds(L, L)]
  lane17 = lax.iota(jnp.int32, L) * (L + 1)

  fire(0)
  bias_cp.wait()
  for j in range(NCH):
    if j + 1 < NCH:
      fire(j + 1)
    drain(j)
    ub = ubuf_v.at[j % 2]
    pb = pbuf_v.at[j % 2]

    def group(g, carry, ub=ub, pb=pb, j=j):
      for r in range(L):
        i = g * L + r
        u0 = ub[i, pl.ds(0, L)] * um0
        u1 = ub[i, pl.ds(L, L)] * um1
        p0 = pb[i, pl.ds(0, L)] * pm0
        p1 = pb[i, pl.ds(L, L)] * pm1
        o = j * CH + i
        concat_v[o, pl.ds(0, L)] = u0
        concat_v[o, pl.ds(L, L)] = u1
        concat_v[o, pl.ds(2 * L, L)] = p0
        concat_v[o, pl.ds(3 * L, L)] = p1
        t = u0 * p0 + u1 * p1
        plsc.store_scatter(tbuf_v, [lane17 + r], t)
      acc = tbuf_v[pl.ds(0, L)]
      for k in range(1, L):
        acc = acc + tbuf_v[pl.ds(k * (L + 1), L)]
      og = j * CH + g * L
      score_v[pl.ds(og, L)] = acc + bias_v[pl.ds(og, L)]
      return carry

    lax.fori_loop(0, CH // L, group, 0)

  pltpu.sync_copy(score_v, score_hbm.at[pl.ds(base, BPW)])
  pltpu.sync_copy(concat_v, concat_hbm.at[pl.ds(base, BPW)])


@jax.jit
def _mvem_sc(uidx, pidx, user_emb, product_emb, product_bias,
             user_emb_mask, product_emb_mask):
  mesh = plsc.VectorSubcoreMesh(
      core_axis_name="c", subcore_axis_name="s", num_cores=NC,
      num_subcores=NS)
  run = pl.kernel(
      _sc_body,
      out_type=(jax.ShapeDtypeStruct((B,), jnp.float32),
                jax.ShapeDtypeStruct((B, 2 * D), jnp.float32)),
      mesh=mesh,
      scratch_types=[
          pltpu.VMEM((BPW,), jnp.int32),
          pltpu.VMEM((BPW,), jnp.int32),
          pltpu.VMEM((2, CH, D), jnp.float32),
          pltpu.VMEM((2, CH, D), jnp.float32),
          pltpu.VMEM((BPW,), jnp.float32),
          pltpu.VMEM((D,), jnp.float32),
          pltpu.VMEM((D,), jnp.float32),
          pltpu.VMEM((BPW, 2 * D), jnp.float32),
          pltpu.VMEM((BPW,), jnp.float32),
          pltpu.VMEM((L * (L + 1),), jnp.float32),
          pltpu.SemaphoreType.DMA,
          pltpu.SemaphoreType.DMA,
          pltpu.SemaphoreType.DMA,
      ],
      compiler_params=pltpu.CompilerParams(
          needs_layout_passes=False, use_tc_tiling_on_sc=True),
  )
  return run(uidx, pidx, user_emb, product_emb, product_bias,
             user_emb_mask, product_emb_mask)


def kernel(user_idx, product_idx, user_emb, product_emb, product_bias,
           user_emb_mask, product_emb_mask):
  score, concat = _mvem_sc(user_idx.astype(jnp.int32),
                           product_idx.astype(jnp.int32),
                           user_emb, product_emb, product_bias,
                           user_emb_mask, product_emb_mask)
  return score, concat
